# initial kernel scaffold (unmeasured)
import jax
import jax.numpy as jnp
from jax import lax
from jax.experimental import pallas as pl
from jax.experimental.pallas import tpu as pltpu

N_DEV = 4
SEQ_PER = 512
SEQ = N_DEV * SEQ_PER
D = 1024
DH = 128
H_LOC = 8
SCALE = 0.08838834764831843


def kernel(x, Wq, Wo, Wk, Wv):
    def body(x_ref, wq_ref, wo_ref, wk_ref, wv_ref, out_ref,
             xg_ref, q_ref, k_ref, v_ref, attn_ref, part_ref, rs_ref,
             ag_send, ag_recv, rs_send, rs_recv):
        my = lax.axis_index("i")
        left = lax.rem(my + N_DEV - 1, N_DEV)
        right = lax.rem(my + 1, N_DEV)

        barrier = pltpu.get_barrier_semaphore()
        for nbr in (left, right):
            pl.semaphore_signal(barrier, inc=1, device_id=(nbr,),
                                device_id_type=pl.DeviceIdType.MESH)
        pl.semaphore_wait(barrier, 2)

        xg_ref[0] = x_ref[0].astype(jnp.bfloat16)
        for h in range(N_DEV - 1):
            rdma = pltpu.make_async_remote_copy(
                src_ref=xg_ref.at[h],
                dst_ref=xg_ref.at[h + 1],
                send_sem=ag_send.at[h],
                recv_sem=ag_recv.at[h],
                device_id=(right,),
                device_id_type=pl.DeviceIdType.MESH,
            )
            rdma.start()
            rdma.wait()

        xf = xg_ref[...].reshape(SEQ, D)
        wq = wq_ref[...].astype(jnp.bfloat16)
        wk = wk_ref[...].astype(jnp.bfloat16)
        wv = wv_ref[...].astype(jnp.bfloat16)
        q_ref[...] = jnp.dot(
            xf, wq, preferred_element_type=jnp.float32).astype(jnp.bfloat16)
        k_ref[...] = jnp.dot(
            xf, wk, preferred_element_type=jnp.float32).astype(jnp.bfloat16)
        v_ref[...] = jnp.dot(
            xf, wv, preferred_element_type=jnp.float32).astype(jnp.bfloat16)

        for h in range(H_LOC):
            sl = slice(h * DH, (h + 1) * DH)
            qh = q_ref[:, sl]
            kh = k_ref[:, sl]
            vh = v_ref[:, sl]
            s = lax.dot_general(
                qh, kh, (((1,), (1,)), ((), ())),
                preferred_element_type=jnp.float32) * SCALE
            mx = jnp.max(s, axis=1, keepdims=True)
            p = jnp.exp(s - mx)
            l = jnp.sum(p, axis=1, keepdims=True)
            o = lax.dot_general(
                p.astype(jnp.bfloat16), vh, (((1,), (0,)), ((), ())),
                preferred_element_type=jnp.float32)
            attn_ref[:, sl] = (o / l).astype(jnp.bfloat16)

        wo = wo_ref[...].astype(jnp.bfloat16)
        part = jnp.dot(
            attn_ref[...], wo, preferred_element_type=jnp.float32)
        part_ref[...] = part.reshape(N_DEV, SEQ_PER, D)

        for s_idx in range(N_DEV - 1):
            src = part_ref.at[1] if s_idx == 0 else rs_ref.at[s_idx - 1]
            rdma = pltpu.make_async_remote_copy(
                src_ref=src,
                dst_ref=rs_ref.at[s_idx],
                send_sem=rs_send.at[s_idx],
                recv_sem=rs_recv.at[s_idx],
                device_id=(right,),
                device_id_type=pl.DeviceIdType.MESH,
            )
            rdma.start()
            rdma.wait()
            if s_idx < N_DEV - 2:
                rs_ref[s_idx] = rs_ref[s_idx] + part_ref[2 + s_idx]

        out_ref[0] = rs_ref[N_DEV - 2] + part_ref[0]

    return pl.pallas_call(
        body,
        out_shape=jax.ShapeDtypeStruct((1, SEQ_PER, D), jnp.float32),
        in_specs=[pl.BlockSpec(memory_space=pltpu.VMEM)] * 5,
        out_specs=pl.BlockSpec(memory_space=pltpu.VMEM),
        scratch_shapes=[
            pltpu.VMEM((N_DEV, SEQ_PER, D), jnp.bfloat16),
            pltpu.VMEM((SEQ, D), jnp.bfloat16),
            pltpu.VMEM((SEQ, D), jnp.bfloat16),
            pltpu.VMEM((SEQ, D), jnp.bfloat16),
            pltpu.VMEM((SEQ, D), jnp.bfloat16),
            pltpu.VMEM((N_DEV, SEQ_PER, D), jnp.float32),
            pltpu.VMEM((N_DEV - 1, SEQ_PER, D), jnp.float32),
            pltpu.SemaphoreType.DMA((N_DEV - 1,)),
            pltpu.SemaphoreType.DMA((N_DEV - 1,)),
            pltpu.SemaphoreType.DMA((N_DEV - 1,)),
            pltpu.SemaphoreType.DMA((N_DEV - 1,)),
        ],
        compiler_params=pltpu.CompilerParams(collective_id=0),
    )(x, Wq, Wo, Wk, Wv)


# baseline (device time: 255739 ns/iter reference)
import jax
import jax.numpy as jnp
from jax import lax
from jax.experimental import pallas as pl
from jax.experimental.pallas import tpu as pltpu

N_DEV = 4
SEQ_PER = 512
SEQ = N_DEV * SEQ_PER
D = 1024
DH = 128
H_LOC = 8
SCALE = 0.08838834764831843


def kernel(x, Wq, Wo, Wk, Wv):
    def body(x_ref, wq_ref, wo_ref, wk_ref, wv_ref, out_ref,
             xg_ref, part_ref, rs_ref,
             ag_send, ag_recv, rs_send, rs_recv):
        my = lax.axis_index("i")
        left = lax.rem(my + N_DEV - 1, N_DEV)
        right = lax.rem(my + 1, N_DEV)

        barrier = pltpu.get_barrier_semaphore()
        for nbr in (left, right):
            pl.semaphore_signal(barrier, inc=1, device_id=(nbr,),
                                device_id_type=pl.DeviceIdType.MESH)
        pl.semaphore_wait(barrier, 2)

        xg_ref[0] = x_ref[0]
        for h in range(N_DEV - 1):
            rdma = pltpu.make_async_remote_copy(
                src_ref=xg_ref.at[h],
                dst_ref=xg_ref.at[h + 1],
                send_sem=ag_send.at[h],
                recv_sem=ag_recv.at[h],
                device_id=(right,),
                device_id_type=pl.DeviceIdType.MESH,
            )
            rdma.start()
            rdma.wait()

        xf = xg_ref[...].reshape(SEQ, D)
        part_ref[...] = jnp.zeros((N_DEV, SEQ_PER, D), jnp.float32)
        for h in range(H_LOC):
            sl = slice(h * DH, (h + 1) * DH)
            kh = jnp.dot(xf, wk_ref[:, sl],
                         preferred_element_type=jnp.float32).astype(jnp.bfloat16)
            vh = jnp.dot(xf, wv_ref[:, sl],
                         preferred_element_type=jnp.float32).astype(jnp.bfloat16)
            woh = wo_ref[sl, :]
            for qb in range(N_DEV):
                qh = jnp.dot(xg_ref[qb], wq_ref[:, sl],
                             preferred_element_type=jnp.float32
                             ).astype(jnp.bfloat16)
                s = lax.dot_general(
                    qh, kh, (((1,), (1,)), ((), ())),
                    preferred_element_type=jnp.float32) * SCALE
                mx = jnp.max(s, axis=1, keepdims=True)
                p = jnp.exp(s - mx)
                l = jnp.sum(p, axis=1, keepdims=True)
                o = lax.dot_general(
                    p.astype(jnp.bfloat16), vh, (((1,), (0,)), ((), ())),
                    preferred_element_type=jnp.float32)
                o_bf = (o / l).astype(jnp.bfloat16)
                part_ref[qb] = part_ref[qb] + jnp.dot(
                    o_bf, woh, preferred_element_type=jnp.float32)

        for s_idx in range(N_DEV - 1):
            src = part_ref.at[1] if s_idx == 0 else rs_ref.at[s_idx - 1]
            rdma = pltpu.make_async_remote_copy(
                src_ref=src,
                dst_ref=rs_ref.at[s_idx],
                send_sem=rs_send.at[s_idx],
                recv_sem=rs_recv.at[s_idx],
                device_id=(right,),
                device_id_type=pl.DeviceIdType.MESH,
            )
            rdma.start()
            rdma.wait()
            if s_idx < N_DEV - 2:
                rs_ref[s_idx] = rs_ref[s_idx] + part_ref[2 + s_idx]

        out_ref[0] = rs_ref[N_DEV - 2] + part_ref[0]

    f = pl.pallas_call(
        body,
        out_shape=jax.ShapeDtypeStruct((1, SEQ_PER, D), jnp.float32),
        in_specs=[pl.BlockSpec(memory_space=pltpu.VMEM)] * 5,
        out_specs=pl.BlockSpec(memory_space=pltpu.VMEM),
        scratch_shapes=[
            pltpu.VMEM((N_DEV, SEQ_PER, D), jnp.bfloat16),
            pltpu.VMEM((N_DEV, SEQ_PER, D), jnp.float32),
            pltpu.VMEM((N_DEV - 1, SEQ_PER, D), jnp.float32),
            pltpu.SemaphoreType.DMA((N_DEV - 1,)),
            pltpu.SemaphoreType.DMA((N_DEV - 1,)),
            pltpu.SemaphoreType.DMA((N_DEV - 1,)),
            pltpu.SemaphoreType.DMA((N_DEV - 1,)),
        ],
        compiler_params=pltpu.CompilerParams(collective_id=0),
    )
    return f(
        x.astype(jnp.bfloat16),
        Wq.astype(jnp.bfloat16),
        Wo.astype(jnp.bfloat16),
        Wk.astype(jnp.bfloat16),
        Wv.astype(jnp.bfloat16),
    )


# device time: 156456 ns/iter; 1.6346x vs baseline; 1.6346x over previous
import jax
import jax.numpy as jnp
from jax import lax
from jax.experimental import pallas as pl
from jax.experimental.pallas import tpu as pltpu

N_DEV = 4
SEQ_PER = 512
SEQ = N_DEV * SEQ_PER
D = 1024
DH = 128
H_LOC = 8
SCALE = 0.08838834764831843


def kernel(x, Wq, Wo, Wk, Wv):
    def body(x_ref, wq_ref, wo_ref, wk_ref, wv_ref, out_ref,
             xg_ref, q_ref, k_ref, v_ref, rs_ref,
             ag_send, ag_recv, rs_send, rs_recv):
        my = lax.axis_index("i")
        left = lax.rem(my + N_DEV - 1, N_DEV)
        right = lax.rem(my + 1, N_DEV)

        barrier = pltpu.get_barrier_semaphore()
        for nbr in (left, right):
            pl.semaphore_signal(barrier, inc=1, device_id=(nbr,),
                                device_id_type=pl.DeviceIdType.MESH)
        pl.semaphore_wait(barrier, 2)

        def qkv_chunk(j):
            xj = xg_ref[j]
            rows = slice(j * SEQ_PER, (j + 1) * SEQ_PER)
            q_ref[rows, :] = jnp.dot(
                xj, wq_ref[...],
                preferred_element_type=jnp.float32).astype(jnp.bfloat16)
            k_ref[rows, :] = jnp.dot(
                xj, wk_ref[...],
                preferred_element_type=jnp.float32).astype(jnp.bfloat16)
            v_ref[rows, :] = jnp.dot(
                xj, wv_ref[...],
                preferred_element_type=jnp.float32).astype(jnp.bfloat16)

        xg_ref[0] = x_ref[0]
        for h in range(N_DEV - 1):
            rdma = pltpu.make_async_remote_copy(
                src_ref=xg_ref.at[h],
                dst_ref=xg_ref.at[h + 1],
                send_sem=ag_send.at[h],
                recv_sem=ag_recv.at[h],
                device_id=(right,),
                device_id_type=pl.DeviceIdType.MESH,
            )
            rdma.start()
            qkv_chunk(h)
            rdma.wait()
        qkv_chunk(N_DEV - 1)

        def part_chunk(qb):
            rows = slice(qb * SEQ_PER, (qb + 1) * SEQ_PER)
            acc = jnp.zeros((SEQ_PER, D), jnp.float32)
            for h in range(H_LOC):
                sl = slice(h * DH, (h + 1) * DH)
                qh = q_ref[rows, sl]
                kh = k_ref[:, sl]
                vh = v_ref[:, sl]
                s = lax.dot_general(
                    qh, kh, (((1,), (1,)), ((), ())),
                    preferred_element_type=jnp.float32) * SCALE
                mx = jnp.max(s, axis=1, keepdims=True)
                p = jnp.exp(s - mx)
                l = jnp.sum(p, axis=1, keepdims=True)
                o = lax.dot_general(
                    p.astype(jnp.bfloat16), vh, (((1,), (0,)), ((), ())),
                    preferred_element_type=jnp.float32)
                acc = acc + jnp.dot(
                    (o / l).astype(jnp.bfloat16), wo_ref[sl, :],
                    preferred_element_type=jnp.float32)
            return acc

        rs_ref[3] = part_chunk(1).astype(jnp.bfloat16)
        nxt = [2, 3, 0]
        part0 = None
        for s_idx in range(N_DEV - 1):
            rdma = pltpu.make_async_remote_copy(
                src_ref=rs_ref.at[3 if s_idx == 0 else s_idx - 1],
                dst_ref=rs_ref.at[s_idx],
                send_sem=rs_send.at[s_idx],
                recv_sem=rs_recv.at[s_idx],
                device_id=(right,),
                device_id_type=pl.DeviceIdType.MESH,
            )
            rdma.start()
            part = part_chunk(nxt[s_idx])
            rdma.wait()
            if s_idx < N_DEV - 2:
                rs_ref[s_idx] = (
                    rs_ref[s_idx].astype(jnp.float32) + part
                ).astype(jnp.bfloat16)
            else:
                part0 = part

        out_ref[0] = rs_ref[N_DEV - 2].astype(jnp.float32) + part0

    f = pl.pallas_call(
        body,
        out_shape=jax.ShapeDtypeStruct((1, SEQ_PER, D), jnp.float32),
        in_specs=[pl.BlockSpec(memory_space=pltpu.VMEM)] * 5,
        out_specs=pl.BlockSpec(memory_space=pltpu.VMEM),
        scratch_shapes=[
            pltpu.VMEM((N_DEV, SEQ_PER, D), jnp.bfloat16),
            pltpu.VMEM((SEQ, D), jnp.bfloat16),
            pltpu.VMEM((SEQ, D), jnp.bfloat16),
            pltpu.VMEM((SEQ, D), jnp.bfloat16),
            pltpu.VMEM((N_DEV, SEQ_PER, D), jnp.bfloat16),
            pltpu.SemaphoreType.DMA((N_DEV - 1,)),
            pltpu.SemaphoreType.DMA((N_DEV - 1,)),
            pltpu.SemaphoreType.DMA((N_DEV - 1,)),
            pltpu.SemaphoreType.DMA((N_DEV - 1,)),
        ],
        compiler_params=pltpu.CompilerParams(collective_id=0),
    )
    return f(
        x.astype(jnp.bfloat16),
        Wq.astype(jnp.bfloat16),
        Wo.astype(jnp.bfloat16),
        Wk.astype(jnp.bfloat16),
        Wv.astype(jnp.bfloat16),
    )
